# baseline (speedup 1.0000x reference)
import functools

import jax
import jax.numpy as jnp
from jax import lax
from jax.experimental import pallas as pl
from jax.experimental.pallas import tpu as pltpu
from jax.experimental.pallas import tpu_sc as plsc

N = 1024
D = 64
M = 100000
BLK = 2048
NB = (M + BLK - 1) // BLK
LANES = 128
CHUNKS = BLK // LANES

NOISE_MIN = 0.01
NOISE_MAX = 0.5

SC_NC = 2
SC_NS = 16
SC_NW = SC_NC * SC_NS
BPW = N // SC_NW


def _tc_body(f_ref, mem_ref, noise_ref, isc_ref, dsc_ref,
             idx_out, ns_out, si_out, bv, bi):
    pid = pl.program_id(0)

    @pl.when(pid == 0)
    def _init():
        bv[...] = jnp.full((N, LANES), jnp.inf, jnp.float32)
        bi[...] = jnp.zeros((N, LANES), jnp.int32)

    f = f_ref[...]
    mem = mem_ref[...]
    m2 = jnp.sum(mem * mem, axis=1, keepdims=True)
    aug = jnp.concatenate([mem, m2], axis=1)
    q = jnp.concatenate([f * -2.0, jnp.ones((N, 1), jnp.float32)], axis=1)
    d2p = lax.dot_general(q, aug, (((1,), (1,)), ((), ())),
                          preferred_element_type=jnp.float32)

    base = pid * BLK
    lane_iota = lax.broadcasted_iota(jnp.int32, (N, LANES), 1)

    def accumulate(masked):
        bv_c = bv[...]
        bi_c = bi[...]
        for c in range(CHUNKS):
            chunk = d2p[:, c * LANES:(c + 1) * LANES]
            idxs = lane_iota + (base + c * LANES)
            mask = chunk < bv_c
            if masked:
                mask = jnp.logical_and(mask, idxs < M)
            bv_c = jnp.where(mask, chunk, bv_c)
            bi_c = jnp.where(mask, idxs, bi_c)
        bv[...] = bv_c
        bi[...] = bi_c

    @pl.when(pid < NB - 1)
    def _full():
        accumulate(False)

    @pl.when(pid == NB - 1)
    def _ragged():
        accumulate(True)

    @pl.when(pid == NB - 1)
    def _fin():
        bv_f = bv[...]
        bi_f = bi[...]
        lane_min = jnp.min(bv_f, axis=1, keepdims=True)
        cand = jnp.where(bv_f == lane_min, bi_f, jnp.int32(2147483647))
        nn = jnp.min(cand, axis=1, keepdims=True)
        x2 = jnp.sum(f * f, axis=1, keepdims=True)
        d2min = jnp.maximum(lane_min + x2, 0.0)
        dist = jnp.sqrt(d2min + 1e-12)
        dsc = dsc_ref[0, 0]
        isc = isc_ref[0, 0]
        nstd = jnp.clip(dsc * dist, NOISE_MIN, NOISE_MAX)
        idx_out[...] = nn
        ns_out[...] = noise_ref[...] * nstd
        si_out[...] = jnp.broadcast_to(isc / (dist + 1e-8), (N, D))


_tc_argmin = pl.pallas_call(
    _tc_body,
    grid=(NB,),
    in_specs=[
        pl.BlockSpec((N, D), lambda i: (0, 0)),
        pl.BlockSpec((BLK, D), lambda i: (i, 0)),
        pl.BlockSpec((N, D), lambda i: (0, 0)),
        pl.BlockSpec(memory_space=pltpu.SMEM),
        pl.BlockSpec(memory_space=pltpu.SMEM),
    ],
    out_specs=[
        pl.BlockSpec((N, 1), lambda i: (0, 0)),
        pl.BlockSpec((N, D), lambda i: (0, 0)),
        pl.BlockSpec((N, D), lambda i: (0, 0)),
    ],
    out_shape=[
        jax.ShapeDtypeStruct((N, 1), jnp.int32),
        jax.ShapeDtypeStruct((N, D), jnp.float32),
        jax.ShapeDtypeStruct((N, D), jnp.float32),
    ],
    scratch_shapes=[
        pltpu.VMEM((N, LANES), jnp.float32),
        pltpu.VMEM((N, LANES), jnp.int32),
    ],
    compiler_params=pltpu.CompilerParams(
        dimension_semantics=("arbitrary",),
    ),
)


@functools.partial(
    pl.kernel,
    out_type=jax.ShapeDtypeStruct((N, D), jnp.float32),
    mesh=plsc.VectorSubcoreMesh(core_axis_name="c", subcore_axis_name="s"),
    scratch_types=[
        pltpu.VMEM((BPW,), jnp.int32),
        pltpu.VMEM((BPW, D), jnp.float32),
        pltpu.VMEM((BPW, D), jnp.float32),
        pltpu.VMEM((BPW, D), jnp.float32),
        pltpu.VMEM((BPW, D), jnp.float32),
        pltpu.VMEM((BPW, D), jnp.float32),
        pltpu.SemaphoreType.DMA,
    ],
    compiler_params=pltpu.CompilerParams(use_tc_tiling_on_sc=False),
)
def _sc_gather_noise(feat_hbm, mem_hbm, idx_hbm, ns_hbm, si_hbm,
                     out_hbm, idx_v, x_v, m_v, ns_v, si_v, o_v, sem):
    wid = lax.axis_index("s") * SC_NC + lax.axis_index("c")
    base = wid * BPW
    pltpu.sync_copy(idx_hbm.at[pl.ds(base, BPW)], idx_v)
    gather = pltpu.async_copy(mem_hbm.at[idx_v], m_v, sem)
    pltpu.sync_copy(feat_hbm.at[pl.ds(base, BPW)], x_v)
    pltpu.sync_copy(ns_hbm.at[pl.ds(base, BPW)], ns_v)
    pltpu.sync_copy(si_hbm.at[pl.ds(base, BPW)], si_v)
    gather.wait()
    for r in range(BPW):
        for c in range(D // 16):
            sl = pl.ds(c * 16, 16)
            x = x_v[r, sl]
            m = m_v[r, sl]
            t = jnp.minimum(jnp.maximum(jnp.abs(x - m) * si_v[r, sl], 0.0), 1.0)
            o_v[r, sl] = x + ns_v[r, sl] * (1.0 - t)
    pltpu.sync_copy(o_v, out_hbm.at[pl.ds(base, BPW)])


def kernel(features, memory_bank, influence_scale, distance_scale):
    noise = jax.random.normal(jax.random.key(1234), (N, D), dtype=jnp.float32)
    isc = jnp.reshape(influence_scale, (1, 1))
    dsc = jnp.reshape(distance_scale, (1, 1))
    nn_idx, noise_scaled, si = _tc_argmin(features, memory_bank, noise, isc, dsc)
    return _sc_gather_noise(features, memory_bank, jnp.reshape(nn_idx, (N,)),
                            noise_scaled, si)


# speedup vs baseline: 1.3550x; 1.3550x over previous
import functools

import jax
import jax.numpy as jnp
from jax import lax
from jax.experimental import pallas as pl
from jax.experimental.pallas import tpu as pltpu
from jax.experimental.pallas import tpu_sc as plsc

N = 1024
D = 64
M = 100000
BLK = 2000
NB = M // BLK
LANES = 128
CHUNKS = 16
RG = 128

NOISE_MIN = 0.01
NOISE_MAX = 0.5

SC_NC = 2
SC_NS = 16
SC_NW = SC_NC * SC_NS
BPW = N // SC_NW


def _tc_body(f_ref, mem_ref, noise_ref, isc_ref, dsc_ref,
             idx_out, ns_out, si_out, bv, bi, q_s):
    pid = pl.program_id(0)

    @pl.when(pid == 0)
    def _init():
        bv[...] = jnp.full((N, LANES), jnp.inf, jnp.float32)
        bi[...] = jnp.zeros((N, LANES), jnp.int32)
        q_s[...] = jnp.concatenate(
            [f_ref[...] * -2.0, jnp.ones((N, 1), jnp.float32)], axis=1)

    mem = mem_ref[...]
    m2 = jnp.sum(mem * mem, axis=1, keepdims=True)
    aug = jnp.concatenate([mem, m2], axis=1)
    d2p = lax.dot_general(q_s[...], aug, (((1,), (1,)), ((), ())),
                          preferred_element_type=jnp.float32)

    base = pid * BLK
    lane_iota = lax.broadcasted_iota(jnp.int32, (RG, LANES), 1)
    tail_pad = jnp.full((RG, CHUNKS * LANES - BLK), jnp.inf, jnp.float32)
    for rg in range(N // RG):
        r0 = rg * RG
        bvs = bv[r0:r0 + RG, :]
        bis = bi[r0:r0 + RG, :]
        for c in range(CHUNKS):
            if (c + 1) * LANES <= BLK:
                chunk = d2p[r0:r0 + RG, c * LANES:(c + 1) * LANES]
            else:
                chunk = jnp.concatenate(
                    [d2p[r0:r0 + RG, c * LANES:BLK], tail_pad], axis=1)
            idxs = lane_iota + (base + c * LANES)
            mask = chunk < bvs
            bvs = jnp.where(mask, chunk, bvs)
            bis = jnp.where(mask, idxs, bis)
        bv[r0:r0 + RG, :] = bvs
        bi[r0:r0 + RG, :] = bis

    @pl.when(pid == NB - 1)
    def _fin():
        bv_f = bv[...]
        bi_f = bi[...]
        lane_min = jnp.min(bv_f, axis=1, keepdims=True)
        cand = jnp.where(bv_f == lane_min, bi_f, jnp.int32(2147483647))
        nn = jnp.min(cand, axis=1, keepdims=True)
        f = f_ref[...]
        x2 = jnp.sum(f * f, axis=1, keepdims=True)
        d2min = jnp.maximum(lane_min + x2, 0.0)
        dist = jnp.sqrt(d2min + 1e-12)
        dsc = dsc_ref[0, 0]
        isc = isc_ref[0, 0]
        nstd = jnp.clip(dsc * dist, NOISE_MIN, NOISE_MAX)
        idx_out[...] = nn
        ns_out[...] = noise_ref[...] * nstd
        si_out[...] = jnp.broadcast_to(isc / (dist + 1e-8), (N, D))


_tc_argmin = pl.pallas_call(
    _tc_body,
    grid=(NB,),
    in_specs=[
        pl.BlockSpec((N, D), lambda i: (0, 0)),
        pl.BlockSpec((BLK, D), lambda i: (i, 0)),
        pl.BlockSpec((N, D), lambda i: (0, 0)),
        pl.BlockSpec(memory_space=pltpu.SMEM),
        pl.BlockSpec(memory_space=pltpu.SMEM),
    ],
    out_specs=[
        pl.BlockSpec((N, 1), lambda i: (0, 0)),
        pl.BlockSpec((N, D), lambda i: (0, 0)),
        pl.BlockSpec((N, D), lambda i: (0, 0)),
    ],
    out_shape=[
        jax.ShapeDtypeStruct((N, 1), jnp.int32),
        jax.ShapeDtypeStruct((N, D), jnp.float32),
        jax.ShapeDtypeStruct((N, D), jnp.float32),
    ],
    scratch_shapes=[
        pltpu.VMEM((N, LANES), jnp.float32),
        pltpu.VMEM((N, LANES), jnp.int32),
        pltpu.VMEM((N, D + 1), jnp.float32),
    ],
    compiler_params=pltpu.CompilerParams(
        dimension_semantics=("arbitrary",),
    ),
)


@functools.partial(
    pl.kernel,
    out_type=jax.ShapeDtypeStruct((N, D), jnp.float32),
    mesh=plsc.VectorSubcoreMesh(core_axis_name="c", subcore_axis_name="s"),
    scratch_types=[
        pltpu.VMEM((BPW,), jnp.int32),
        pltpu.VMEM((BPW, D), jnp.float32),
        pltpu.VMEM((BPW, D), jnp.float32),
        pltpu.VMEM((BPW, D), jnp.float32),
        pltpu.VMEM((BPW, D), jnp.float32),
        pltpu.VMEM((BPW, D), jnp.float32),
        pltpu.SemaphoreType.DMA,
    ],
    compiler_params=pltpu.CompilerParams(use_tc_tiling_on_sc=False),
)
def _sc_gather_noise(feat_hbm, mem_hbm, idx_hbm, ns_hbm, si_hbm,
                     out_hbm, idx_v, x_v, m_v, ns_v, si_v, o_v, sem):
    wid = lax.axis_index("s") * SC_NC + lax.axis_index("c")
    base = wid * BPW
    pltpu.sync_copy(idx_hbm.at[pl.ds(base, BPW)], idx_v)
    gather = pltpu.async_copy(mem_hbm.at[idx_v], m_v, sem)
    pltpu.sync_copy(feat_hbm.at[pl.ds(base, BPW)], x_v)
    pltpu.sync_copy(ns_hbm.at[pl.ds(base, BPW)], ns_v)
    pltpu.sync_copy(si_hbm.at[pl.ds(base, BPW)], si_v)
    gather.wait()
    for r in range(BPW):
        for c in range(D // 16):
            sl = pl.ds(c * 16, 16)
            x = x_v[r, sl]
            m = m_v[r, sl]
            t = jnp.minimum(jnp.maximum(jnp.abs(x - m) * si_v[r, sl], 0.0), 1.0)
            o_v[r, sl] = x + ns_v[r, sl] * (1.0 - t)
    pltpu.sync_copy(o_v, out_hbm.at[pl.ds(base, BPW)])


def kernel(features, memory_bank, influence_scale, distance_scale):
    noise = jax.random.normal(jax.random.key(1234), (N, D), dtype=jnp.float32)
    isc = jnp.reshape(influence_scale, (1, 1))
    dsc = jnp.reshape(distance_scale, (1, 1))
    nn_idx, noise_scaled, si = _tc_argmin(features, memory_bank, noise, isc, dsc)
    return _sc_gather_noise(features, memory_bank, jnp.reshape(nn_idx, (N,)),
                            noise_scaled, si)


# speedup vs baseline: 1.4153x; 1.0445x over previous
import functools

import jax
import jax.numpy as jnp
from jax import lax
from jax.experimental import pallas as pl
from jax.experimental.pallas import tpu as pltpu
from jax.experimental.pallas import tpu_sc as plsc

N = 1024
D = 64
M = 100000
BLK = 4000
NB = M // BLK
LANES = 128
CHUNKS = 32
RG = 128

NOISE_MIN = 0.01
NOISE_MAX = 0.5

SC_NC = 2
SC_NS = 16
SC_NW = SC_NC * SC_NS
BPW = N // SC_NW


def _tc_body(f_ref, mem_ref, noise_ref, isc_ref, dsc_ref,
             idx_out, ns_out, si_out, bv, bi, q_s):
    pid = pl.program_id(0)

    @pl.when(pid == 0)
    def _init():
        bv[...] = jnp.full((N, LANES), jnp.inf, jnp.float32)
        bi[...] = jnp.zeros((N, LANES), jnp.int32)
        q_s[...] = jnp.concatenate(
            [f_ref[...] * -2.0, jnp.ones((N, 1), jnp.float32)], axis=1)

    mem = mem_ref[...]
    m2 = jnp.sum(mem * mem, axis=1, keepdims=True)
    aug = jnp.concatenate([mem, m2], axis=1)
    d2p = lax.dot_general(q_s[...], aug, (((1,), (1,)), ((), ())),
                          preferred_element_type=jnp.float32)

    base = pid * BLK
    lane_iota = lax.broadcasted_iota(jnp.int32, (RG, LANES), 1)
    tail_pad = jnp.full((RG, CHUNKS * LANES - BLK), jnp.inf, jnp.float32)
    for rg in range(N // RG):
        r0 = rg * RG
        bvs = bv[r0:r0 + RG, :]
        bis = bi[r0:r0 + RG, :]
        for c in range(CHUNKS):
            if (c + 1) * LANES <= BLK:
                chunk = d2p[r0:r0 + RG, c * LANES:(c + 1) * LANES]
            else:
                chunk = jnp.concatenate(
                    [d2p[r0:r0 + RG, c * LANES:BLK], tail_pad], axis=1)
            idxs = lane_iota + (base + c * LANES)
            mask = chunk < bvs
            bvs = jnp.where(mask, chunk, bvs)
            bis = jnp.where(mask, idxs, bis)
        bv[r0:r0 + RG, :] = bvs
        bi[r0:r0 + RG, :] = bis

    @pl.when(pid == NB - 1)
    def _fin():
        bv_f = bv[...]
        bi_f = bi[...]
        lane_min = jnp.min(bv_f, axis=1, keepdims=True)
        cand = jnp.where(bv_f == lane_min, bi_f, jnp.int32(2147483647))
        nn = jnp.min(cand, axis=1, keepdims=True)
        f = f_ref[...]
        x2 = jnp.sum(f * f, axis=1, keepdims=True)
        d2min = jnp.maximum(lane_min + x2, 0.0)
        dist = jnp.sqrt(d2min + 1e-12)
        dsc = dsc_ref[0, 0]
        isc = isc_ref[0, 0]
        nstd = jnp.clip(dsc * dist, NOISE_MIN, NOISE_MAX)
        idx_out[...] = nn
        ns_out[...] = noise_ref[...] * nstd
        si_out[...] = jnp.broadcast_to(isc / (dist + 1e-8), (N, D))


_tc_argmin = pl.pallas_call(
    _tc_body,
    grid=(NB,),
    in_specs=[
        pl.BlockSpec((N, D), lambda i: (0, 0)),
        pl.BlockSpec((BLK, D), lambda i: (i, 0)),
        pl.BlockSpec((N, D), lambda i: (0, 0)),
        pl.BlockSpec(memory_space=pltpu.SMEM),
        pl.BlockSpec(memory_space=pltpu.SMEM),
    ],
    out_specs=[
        pl.BlockSpec((N, 1), lambda i: (0, 0)),
        pl.BlockSpec((N, D), lambda i: (0, 0)),
        pl.BlockSpec((N, D), lambda i: (0, 0)),
    ],
    out_shape=[
        jax.ShapeDtypeStruct((N, 1), jnp.int32),
        jax.ShapeDtypeStruct((N, D), jnp.float32),
        jax.ShapeDtypeStruct((N, D), jnp.float32),
    ],
    scratch_shapes=[
        pltpu.VMEM((N, LANES), jnp.float32),
        pltpu.VMEM((N, LANES), jnp.int32),
        pltpu.VMEM((N, D + 1), jnp.float32),
    ],
    compiler_params=pltpu.CompilerParams(
        dimension_semantics=("arbitrary",),
    ),
)


@functools.partial(
    pl.kernel,
    out_type=jax.ShapeDtypeStruct((N, D), jnp.float32),
    mesh=plsc.VectorSubcoreMesh(core_axis_name="c", subcore_axis_name="s"),
    scratch_types=[
        pltpu.VMEM((BPW,), jnp.int32),
        pltpu.VMEM((BPW, D), jnp.float32),
        pltpu.VMEM((BPW, D), jnp.float32),
        pltpu.VMEM((BPW, D), jnp.float32),
        pltpu.VMEM((BPW, D), jnp.float32),
        pltpu.VMEM((BPW, D), jnp.float32),
        pltpu.SemaphoreType.DMA,
    ],
    compiler_params=pltpu.CompilerParams(use_tc_tiling_on_sc=False),
)
def _sc_gather_noise(feat_hbm, mem_hbm, idx_hbm, ns_hbm, si_hbm,
                     out_hbm, idx_v, x_v, m_v, ns_v, si_v, o_v, sem):
    wid = lax.axis_index("s") * SC_NC + lax.axis_index("c")
    base = wid * BPW
    pltpu.sync_copy(idx_hbm.at[pl.ds(base, BPW)], idx_v)
    gather = pltpu.async_copy(mem_hbm.at[idx_v], m_v, sem)
    pltpu.sync_copy(feat_hbm.at[pl.ds(base, BPW)], x_v)
    pltpu.sync_copy(ns_hbm.at[pl.ds(base, BPW)], ns_v)
    pltpu.sync_copy(si_hbm.at[pl.ds(base, BPW)], si_v)
    gather.wait()
    for r in range(BPW):
        for c in range(D // 16):
            sl = pl.ds(c * 16, 16)
            x = x_v[r, sl]
            m = m_v[r, sl]
            t = jnp.minimum(jnp.maximum(jnp.abs(x - m) * si_v[r, sl], 0.0), 1.0)
            o_v[r, sl] = x + ns_v[r, sl] * (1.0 - t)
    pltpu.sync_copy(o_v, out_hbm.at[pl.ds(base, BPW)])


def kernel(features, memory_bank, influence_scale, distance_scale):
    noise = jax.random.normal(jax.random.key(1234), (N, D), dtype=jnp.float32)
    isc = jnp.reshape(influence_scale, (1, 1))
    dsc = jnp.reshape(distance_scale, (1, 1))
    nn_idx, noise_scaled, si = _tc_argmin(features, memory_bank, noise, isc, dsc)
    return _sc_gather_noise(features, memory_bank, jnp.reshape(nn_idx, (N,)),
                            noise_scaled, si)


# speedup vs baseline: 1.4628x; 1.0336x over previous
import functools

import jax
import jax.numpy as jnp
import numpy as np
from jax import lax
from jax.experimental import pallas as pl
from jax.experimental.pallas import tpu as pltpu
from jax.experimental.pallas import tpu_sc as plsc

N = 1024
D = 64
M = 100000
BLK = 4000
NB = M // BLK
LANES = 128
CHUNKS = 32
RG = 128

NOISE_MIN = 0.01
NOISE_MAX = 0.5

SC_NC = 2
SC_NS = 16
SC_NW = SC_NC * SC_NS
BPW = N // SC_NW


def _tc_body(f_ref, mem_ref, noise_ref, isc_ref, dsc_ref,
             idx_out, ns_out, si_out, par_out, bv, bi, q_s):
    pid = pl.program_id(0)

    @pl.when(pid == 0)
    def _init():
        bv[...] = jnp.full((N, LANES), jnp.inf, jnp.float32)
        bi[...] = jnp.zeros((N, LANES), jnp.int32)
        q_s[...] = jnp.concatenate(
            [f_ref[...] * -2.0, jnp.ones((N, 1), jnp.float32)], axis=1)

    mem = mem_ref[...]
    m2 = jnp.sum(mem * mem, axis=1, keepdims=True)
    aug = jnp.concatenate([mem, m2], axis=1)
    d2p = lax.dot_general(q_s[...], aug, (((1,), (1,)), ((), ())),
                          preferred_element_type=jnp.float32)

    base = pid * BLK
    lane_iota = lax.broadcasted_iota(jnp.int32, (RG, LANES), 1)
    tail_pad = jnp.full((RG, CHUNKS * LANES - BLK), jnp.inf, jnp.float32)
    for rg in range(N // RG):
        r0 = rg * RG
        bvs = bv[r0:r0 + RG, :]
        bis = bi[r0:r0 + RG, :]
        for c in range(CHUNKS):
            if (c + 1) * LANES <= BLK:
                chunk = d2p[r0:r0 + RG, c * LANES:(c + 1) * LANES]
            else:
                chunk = jnp.concatenate(
                    [d2p[r0:r0 + RG, c * LANES:BLK], tail_pad], axis=1)
            idxs = lane_iota + (base + c * LANES)
            mask = chunk < bvs
            bvs = jnp.where(mask, chunk, bvs)
            bis = jnp.where(mask, idxs, bis)
        bv[r0:r0 + RG, :] = bvs
        bi[r0:r0 + RG, :] = bis

    @pl.when(pid == NB - 1)
    def _fin():
        bv_f = bv[...]
        bi_f = bi[...]
        lane_min = jnp.min(bv_f, axis=1, keepdims=True)
        cand = jnp.where(bv_f == lane_min, bi_f, jnp.int32(2147483647))
        nn = jnp.min(cand, axis=1, keepdims=True)
        f = f_ref[...]
        x2 = jnp.sum(f * f, axis=1, keepdims=True)
        d2min = jnp.maximum(lane_min + x2, 0.0)
        dist = jnp.sqrt(d2min + 1e-12)
        dsc = dsc_ref[0, 0]
        isc = isc_ref[0, 0]
        nstd = jnp.clip(dsc * dist, NOISE_MIN, NOISE_MAX)
        idx_out[...] = nn >> 1
        ns_out[...] = noise_ref[...] * nstd
        si_out[...] = jnp.broadcast_to(isc / (dist + 1e-8), (N, D))
        par_out[...] = jnp.broadcast_to((nn & 1).astype(jnp.float32), (N, D))


_tc_argmin = pl.pallas_call(
    _tc_body,
    grid=(NB,),
    in_specs=[
        pl.BlockSpec((N, D), lambda i: (0, 0)),
        pl.BlockSpec((BLK, D), lambda i: (i, 0)),
        pl.BlockSpec((N, D), lambda i: (0, 0)),
        pl.BlockSpec(memory_space=pltpu.SMEM),
        pl.BlockSpec(memory_space=pltpu.SMEM),
    ],
    out_specs=[
        pl.BlockSpec((N, 1), lambda i: (0, 0)),
        pl.BlockSpec((N, D), lambda i: (0, 0)),
        pl.BlockSpec((N, D), lambda i: (0, 0)),
        pl.BlockSpec((N, D), lambda i: (0, 0)),
    ],
    out_shape=[
        jax.ShapeDtypeStruct((N, 1), jnp.int32),
        jax.ShapeDtypeStruct((N, D), jnp.float32),
        jax.ShapeDtypeStruct((N, D), jnp.float32),
        jax.ShapeDtypeStruct((N, D), jnp.float32),
    ],
    scratch_shapes=[
        pltpu.VMEM((N, LANES), jnp.float32),
        pltpu.VMEM((N, LANES), jnp.int32),
        pltpu.VMEM((N, D + 1), jnp.float32),
    ],
    compiler_params=pltpu.CompilerParams(
        dimension_semantics=("arbitrary",),
    ),
)


@functools.partial(
    pl.kernel,
    out_type=jax.ShapeDtypeStruct((N, D), jnp.float32),
    mesh=plsc.VectorSubcoreMesh(core_axis_name="c", subcore_axis_name="s"),
    scratch_types=[
        pltpu.VMEM((BPW,), jnp.int32),
        pltpu.VMEM((BPW, D), jnp.float32),
        pltpu.VMEM((BPW, 2 * D), jnp.float32),
        pltpu.VMEM((BPW, D), jnp.float32),
        pltpu.VMEM((BPW, D), jnp.float32),
        pltpu.VMEM((BPW, D), jnp.float32),
        pltpu.VMEM((BPW, D), jnp.float32),
        pltpu.SemaphoreType.DMA,
    ],
)
def _sc_gather_noise(feat_hbm, mem2_hbm, idx_hbm, ns_hbm, si_hbm, par_hbm,
                     out_hbm, idx_v, x_v, m2_v, ns_v, si_v, par_v, o_v, sem):
    wid = lax.axis_index("s") * SC_NC + lax.axis_index("c")
    base = wid * BPW
    pltpu.sync_copy(idx_hbm.at[pl.ds(base, BPW)], idx_v)
    gather = pltpu.async_copy(mem2_hbm.at[idx_v], m2_v, sem)
    pltpu.sync_copy(feat_hbm.at[pl.ds(base, BPW)], x_v)
    pltpu.sync_copy(ns_hbm.at[pl.ds(base, BPW)], ns_v)
    pltpu.sync_copy(si_hbm.at[pl.ds(base, BPW)], si_v)
    pltpu.sync_copy(par_hbm.at[pl.ds(base, BPW)], par_v)
    gather.wait()
    for r in range(BPW):
        for c in range(D // 16):
            sl = pl.ds(c * 16, 16)
            lo = m2_v[r, pl.ds(c * 16, 16)]
            hi = m2_v[r, pl.ds(D + c * 16, 16)]
            m = jnp.where(par_v[r, sl] != 0.0, hi, lo)
            x = x_v[r, sl]
            t = jnp.minimum(jnp.maximum(jnp.abs(x - m) * si_v[r, sl], 0.0), 1.0)
            o_v[r, sl] = x + ns_v[r, sl] * (1.0 - t)
    pltpu.sync_copy(o_v, out_hbm.at[pl.ds(base, BPW)])


def _fixed_noise():
    return jax.random.normal(jax.random.key(1234), (N, D), dtype=jnp.float32)


try:
    _NOISE = np.asarray(_fixed_noise())
except Exception:
    _NOISE = None


def kernel(features, memory_bank, influence_scale, distance_scale):
    isc = jnp.reshape(influence_scale, (1, 1))
    dsc = jnp.reshape(distance_scale, (1, 1))
    noise = _NOISE if _NOISE is not None else _fixed_noise()
    idx_half, noise_scaled, si, par = _tc_argmin(
        features, memory_bank, noise, isc, dsc)
    mem_pairs = jnp.reshape(memory_bank, (M // 2, 2 * D))
    return _sc_gather_noise(features, mem_pairs, jnp.reshape(idx_half, (N,)),
                            noise_scaled, si, par)


# speedup vs baseline: 1.8690x; 1.2777x over previous
import functools

import jax
import jax.numpy as jnp
import numpy as np
from jax import lax
from jax.experimental import pallas as pl
from jax.experimental.pallas import tpu as pltpu
from jax.experimental.pallas import tpu_sc as plsc

N = 1024
D = 64
M = 100000
HALF = 50000
BLK = 2000
NB = HALF // BLK
LANES = 128
CHUNKS = 16
RG = 128

NOISE_MIN = 0.01
NOISE_MAX = 0.5

SC_NC = 2
SC_NS = 16
SC_NW = SC_NC * SC_NS
BPW = N // SC_NW


def _tc_body(f_ref, memA_ref, memB_ref, noise_ref, isc_ref, dsc_ref,
             idx_out, ns_out, si_out, par_out, pairs_out, bv, bi, q_s):
    pid = pl.program_id(0)

    @pl.when(pid == 0)
    def _init():
        bv[...] = jnp.full((N, LANES), jnp.inf, jnp.float32)
        bi[...] = jnp.zeros((N, LANES), jnp.int32)
        q_s[...] = jnp.concatenate(
            [f_ref[...] * -2.0, jnp.ones((N, 1), jnp.float32)], axis=1)

    memA = memA_ref[...]
    memB = memB_ref[...]
    pairs_out[...] = jnp.concatenate([memA, memB], axis=1)

    q = q_s[...]
    lane_iota = lax.broadcasted_iota(jnp.int32, (RG, LANES), 1)
    tail_pad = jnp.full((RG, CHUNKS * LANES - BLK), jnp.inf, jnp.float32)

    for mem, base in ((memA, pid * BLK), (memB, HALF + pid * BLK)):
        m2 = jnp.sum(mem * mem, axis=1, keepdims=True)
        aug = jnp.concatenate([mem, m2], axis=1)
        d2p = lax.dot_general(q, aug, (((1,), (1,)), ((), ())),
                              preferred_element_type=jnp.float32)
        for rg in range(N // RG):
            r0 = rg * RG
            bvs = bv[r0:r0 + RG, :]
            bis = bi[r0:r0 + RG, :]
            for c in range(CHUNKS):
                if (c + 1) * LANES <= BLK:
                    chunk = d2p[r0:r0 + RG, c * LANES:(c + 1) * LANES]
                else:
                    chunk = jnp.concatenate(
                        [d2p[r0:r0 + RG, c * LANES:BLK], tail_pad], axis=1)
                idxs = lane_iota + (base + c * LANES)
                mask = chunk < bvs
                bvs = jnp.where(mask, chunk, bvs)
                bis = jnp.where(mask, idxs, bis)
            bv[r0:r0 + RG, :] = bvs
            bi[r0:r0 + RG, :] = bis

    @pl.when(pid == NB - 1)
    def _fin():
        bv_f = bv[...]
        bi_f = bi[...]
        lane_min = jnp.min(bv_f, axis=1, keepdims=True)
        cand = jnp.where(bv_f == lane_min, bi_f, jnp.int32(2147483647))
        nn = jnp.min(cand, axis=1, keepdims=True)
        f = f_ref[...]
        x2 = jnp.sum(f * f, axis=1, keepdims=True)
        d2min = jnp.maximum(lane_min + x2, 0.0)
        dist = jnp.sqrt(d2min + 1e-12)
        dsc = dsc_ref[0, 0]
        isc = isc_ref[0, 0]
        nstd = jnp.clip(dsc * dist, NOISE_MIN, NOISE_MAX)
        in_hi = nn >= HALF
        idx_out[...] = jnp.where(in_hi, nn - HALF, nn)
        ns_out[...] = noise_ref[...] * nstd
        si_out[...] = jnp.broadcast_to(isc / (dist + 1e-8), (N, D))
        par_out[...] = jnp.broadcast_to(in_hi.astype(jnp.float32), (N, D))


_tc_argmin = pl.pallas_call(
    _tc_body,
    grid=(NB,),
    in_specs=[
        pl.BlockSpec((N, D), lambda i: (0, 0)),
        pl.BlockSpec((BLK, D), lambda i: (i, 0)),
        pl.BlockSpec((BLK, D), lambda i: (i + NB, 0)),
        pl.BlockSpec((N, D), lambda i: (0, 0)),
        pl.BlockSpec(memory_space=pltpu.SMEM),
        pl.BlockSpec(memory_space=pltpu.SMEM),
    ],
    out_specs=[
        pl.BlockSpec((N, 1), lambda i: (0, 0)),
        pl.BlockSpec((N, D), lambda i: (0, 0)),
        pl.BlockSpec((N, D), lambda i: (0, 0)),
        pl.BlockSpec((N, D), lambda i: (0, 0)),
        pl.BlockSpec((BLK, 2 * D), lambda i: (i, 0)),
    ],
    out_shape=[
        jax.ShapeDtypeStruct((N, 1), jnp.int32),
        jax.ShapeDtypeStruct((N, D), jnp.float32),
        jax.ShapeDtypeStruct((N, D), jnp.float32),
        jax.ShapeDtypeStruct((N, D), jnp.float32),
        jax.ShapeDtypeStruct((HALF, 2 * D), jnp.float32),
    ],
    scratch_shapes=[
        pltpu.VMEM((N, LANES), jnp.float32),
        pltpu.VMEM((N, LANES), jnp.int32),
        pltpu.VMEM((N, D + 1), jnp.float32),
    ],
    compiler_params=pltpu.CompilerParams(
        dimension_semantics=("arbitrary",),
    ),
)


@functools.partial(
    pl.kernel,
    out_type=jax.ShapeDtypeStruct((N, D), jnp.float32),
    mesh=plsc.VectorSubcoreMesh(core_axis_name="c", subcore_axis_name="s"),
    scratch_types=[
        pltpu.VMEM((BPW,), jnp.int32),
        pltpu.VMEM((BPW, D), jnp.float32),
        pltpu.VMEM((BPW, 2 * D), jnp.float32),
        pltpu.VMEM((BPW, D), jnp.float32),
        pltpu.VMEM((BPW, D), jnp.float32),
        pltpu.VMEM((BPW, D), jnp.float32),
        pltpu.VMEM((BPW, D), jnp.float32),
        pltpu.SemaphoreType.DMA,
    ],
)
def _sc_gather_noise(feat_hbm, mem2_hbm, idx_hbm, ns_hbm, si_hbm, par_hbm,
                     out_hbm, idx_v, x_v, m2_v, ns_v, si_v, par_v, o_v, sem):
    wid = lax.axis_index("s") * SC_NC + lax.axis_index("c")
    base = wid * BPW
    pltpu.sync_copy(idx_hbm.at[pl.ds(base, BPW)], idx_v)
    gather = pltpu.async_copy(mem2_hbm.at[idx_v], m2_v, sem)
    pltpu.sync_copy(feat_hbm.at[pl.ds(base, BPW)], x_v)
    pltpu.sync_copy(ns_hbm.at[pl.ds(base, BPW)], ns_v)
    pltpu.sync_copy(si_hbm.at[pl.ds(base, BPW)], si_v)
    pltpu.sync_copy(par_hbm.at[pl.ds(base, BPW)], par_v)
    gather.wait()
    for r in range(BPW):
        for c in range(D // 16):
            sl = pl.ds(c * 16, 16)
            lo = m2_v[r, pl.ds(c * 16, 16)]
            hi = m2_v[r, pl.ds(D + c * 16, 16)]
            m = jnp.where(par_v[r, sl] != 0.0, hi, lo)
            x = x_v[r, sl]
            t = jnp.minimum(jnp.maximum(jnp.abs(x - m) * si_v[r, sl], 0.0), 1.0)
            o_v[r, sl] = x + ns_v[r, sl] * (1.0 - t)
    pltpu.sync_copy(o_v, out_hbm.at[pl.ds(base, BPW)])


def _fixed_noise():
    return jax.random.normal(jax.random.key(1234), (N, D), dtype=jnp.float32)


try:
    _NOISE = np.asarray(_fixed_noise())
except Exception:
    _NOISE = None


def kernel(features, memory_bank, influence_scale, distance_scale):
    isc = jnp.reshape(influence_scale, (1, 1))
    dsc = jnp.reshape(distance_scale, (1, 1))
    noise = _NOISE if _NOISE is not None else _fixed_noise()
    idx_half, noise_scaled, si, par, mem_pairs = _tc_argmin(
        features, memory_bank, memory_bank, noise, isc, dsc)
    return _sc_gather_noise(features, mem_pairs, jnp.reshape(idx_half, (N,)),
                            noise_scaled, si, par)


# speedup vs baseline: 1.8744x; 1.0029x over previous
import functools

import jax
import jax.numpy as jnp
import numpy as np
from jax import lax
from jax.experimental import pallas as pl
from jax.experimental.pallas import tpu as pltpu
from jax.experimental.pallas import tpu_sc as plsc

N = 1024
D = 64
M = 100000
BLK = 4000
NB = M // BLK
PB = BLK // 2
LANES = 128
CHUNKS = 32
RG = 128

NOISE_MIN = 0.01
NOISE_MAX = 0.5

SC_NC = 2
SC_NS = 16
SC_NW = SC_NC * SC_NS
BPW = N // SC_NW


def _tc_body(f_ref, mem_ref, noise_ref, isc_ref, dsc_ref,
             idx_out, ns_out, si_out, par_out, pairs_out, bv, bi, q_s):
    pid = pl.program_id(0)

    @pl.when(pid == 0)
    def _init():
        bv[...] = jnp.full((N, LANES), jnp.inf, jnp.float32)
        bi[...] = jnp.zeros((N, LANES), jnp.int32)
        q_s[...] = jnp.concatenate(
            [f_ref[...] * -2.0, jnp.ones((N, 1), jnp.float32)], axis=1)

    mem = mem_ref[...]
    pairs_out[...] = jnp.concatenate([mem[:PB, :], mem[PB:, :]], axis=1)

    m2 = jnp.sum(mem * mem, axis=1, keepdims=True)
    aug = jnp.concatenate([mem, m2], axis=1)
    d2p = lax.dot_general(q_s[...], aug, (((1,), (1,)), ((), ())),
                          preferred_element_type=jnp.float32)

    base = pid * BLK
    lane_iota = lax.broadcasted_iota(jnp.int32, (RG, LANES), 1)
    tail_pad = jnp.full((RG, CHUNKS * LANES - BLK), jnp.inf, jnp.float32)
    for rg in range(N // RG):
        r0 = rg * RG
        bvs = bv[r0:r0 + RG, :]
        bis = bi[r0:r0 + RG, :]
        for c in range(CHUNKS):
            if (c + 1) * LANES <= BLK:
                chunk = d2p[r0:r0 + RG, c * LANES:(c + 1) * LANES]
            else:
                chunk = jnp.concatenate(
                    [d2p[r0:r0 + RG, c * LANES:BLK], tail_pad], axis=1)
            idxs = lane_iota + (base + c * LANES)
            mask = chunk < bvs
            bvs = jnp.where(mask, chunk, bvs)
            bis = jnp.where(mask, idxs, bis)
        bv[r0:r0 + RG, :] = bvs
        bi[r0:r0 + RG, :] = bis

    @pl.when(pid == NB - 1)
    def _fin():
        bv_f = bv[...]
        bi_f = bi[...]
        lane_min = jnp.min(bv_f, axis=1, keepdims=True)
        cand = jnp.where(bv_f == lane_min, bi_f, jnp.int32(2147483647))
        nn = jnp.min(cand, axis=1, keepdims=True)
        f = f_ref[...]
        x2 = jnp.sum(f * f, axis=1, keepdims=True)
        d2min = jnp.maximum(lane_min + x2, 0.0)
        dist = jnp.sqrt(d2min + 1e-12)
        dsc = dsc_ref[0, 0]
        isc = isc_ref[0, 0]
        nstd = jnp.clip(dsc * dist, NOISE_MIN, NOISE_MAX)
        blk = nn // BLK
        rem = nn - blk * BLK
        in_hi = rem >= PB
        idx_out[...] = blk * PB + jnp.where(in_hi, rem - PB, rem)
        ns_out[...] = noise_ref[...] * nstd
        si_out[...] = jnp.broadcast_to(isc / (dist + 1e-8), (N, D))
        par_out[...] = jnp.broadcast_to(in_hi.astype(jnp.float32), (N, D))


_tc_argmin = pl.pallas_call(
    _tc_body,
    grid=(NB,),
    in_specs=[
        pl.BlockSpec((N, D), lambda i: (0, 0)),
        pl.BlockSpec((BLK, D), lambda i: (i, 0)),
        pl.BlockSpec((N, D), lambda i: (0, 0)),
        pl.BlockSpec(memory_space=pltpu.SMEM),
        pl.BlockSpec(memory_space=pltpu.SMEM),
    ],
    out_specs=[
        pl.BlockSpec((N, 1), lambda i: (0, 0)),
        pl.BlockSpec((N, D), lambda i: (0, 0)),
        pl.BlockSpec((N, D), lambda i: (0, 0)),
        pl.BlockSpec((N, D), lambda i: (0, 0)),
        pl.BlockSpec((PB, 2 * D), lambda i: (i, 0)),
    ],
    out_shape=[
        jax.ShapeDtypeStruct((N, 1), jnp.int32),
        jax.ShapeDtypeStruct((N, D), jnp.float32),
        jax.ShapeDtypeStruct((N, D), jnp.float32),
        jax.ShapeDtypeStruct((N, D), jnp.float32),
        jax.ShapeDtypeStruct((M // 2, 2 * D), jnp.float32),
    ],
    scratch_shapes=[
        pltpu.VMEM((N, LANES), jnp.float32),
        pltpu.VMEM((N, LANES), jnp.int32),
        pltpu.VMEM((N, D + 1), jnp.float32),
    ],
    compiler_params=pltpu.CompilerParams(
        dimension_semantics=("arbitrary",),
    ),
)


@functools.partial(
    pl.kernel,
    out_type=jax.ShapeDtypeStruct((N, D), jnp.float32),
    mesh=plsc.VectorSubcoreMesh(core_axis_name="c", subcore_axis_name="s"),
    scratch_types=[
        pltpu.VMEM((BPW,), jnp.int32),
        pltpu.VMEM((BPW, D), jnp.float32),
        pltpu.VMEM((BPW, 2 * D), jnp.float32),
        pltpu.VMEM((BPW, D), jnp.float32),
        pltpu.VMEM((BPW, D), jnp.float32),
        pltpu.VMEM((BPW, D), jnp.float32),
        pltpu.VMEM((BPW, D), jnp.float32),
        pltpu.SemaphoreType.DMA,
    ],
)
def _sc_gather_noise(feat_hbm, mem2_hbm, idx_hbm, ns_hbm, si_hbm, par_hbm,
                     out_hbm, idx_v, x_v, m2_v, ns_v, si_v, par_v, o_v, sem):
    wid = lax.axis_index("s") * SC_NC + lax.axis_index("c")
    base = wid * BPW
    pltpu.sync_copy(idx_hbm.at[pl.ds(base, BPW)], idx_v)
    gather = pltpu.async_copy(mem2_hbm.at[idx_v], m2_v, sem)
    pltpu.sync_copy(feat_hbm.at[pl.ds(base, BPW)], x_v)
    pltpu.sync_copy(ns_hbm.at[pl.ds(base, BPW)], ns_v)
    pltpu.sync_copy(si_hbm.at[pl.ds(base, BPW)], si_v)
    pltpu.sync_copy(par_hbm.at[pl.ds(base, BPW)], par_v)
    gather.wait()
    for r in range(BPW):
        for c in range(D // 16):
            sl = pl.ds(c * 16, 16)
            lo = m2_v[r, pl.ds(c * 16, 16)]
            hi = m2_v[r, pl.ds(D + c * 16, 16)]
            m = jnp.where(par_v[r, sl] != 0.0, hi, lo)
            x = x_v[r, sl]
            t = jnp.minimum(jnp.maximum(jnp.abs(x - m) * si_v[r, sl], 0.0), 1.0)
            o_v[r, sl] = x + ns_v[r, sl] * (1.0 - t)
    pltpu.sync_copy(o_v, out_hbm.at[pl.ds(base, BPW)])


def _fixed_noise():
    return jax.random.normal(jax.random.key(1234), (N, D), dtype=jnp.float32)


try:
    _NOISE = np.asarray(_fixed_noise())
except Exception:
    _NOISE = None


def kernel(features, memory_bank, influence_scale, distance_scale):
    isc = jnp.reshape(influence_scale, (1, 1))
    dsc = jnp.reshape(distance_scale, (1, 1))
    noise = _NOISE if _NOISE is not None else _fixed_noise()
    idx_half, noise_scaled, si, par, mem_pairs = _tc_argmin(
        features, memory_bank, noise, isc, dsc)
    return _sc_gather_noise(features, mem_pairs, jnp.reshape(idx_half, (N,)),
                            noise_scaled, si, par)
